# Initial kernel scaffold; baseline (speedup 1.0000x reference)
import functools

import jax
import jax.numpy as jnp
from jax import lax
from jax.experimental import pallas as pl
from jax.experimental.pallas import tpu as pltpu
from jax.experimental.pallas import tpu_sc as plsc

N = 10000
E = 320000
F = 128
G = 64
N_OUT = 10

NC = 2
NS = 16
NW = NC * NS

IDXW = 128
PADROWS = -(-E // (NW * IDXW)) * NW
EPAD = PADROWS * IDXW
RPW = PADROWS // NW
PADN = 10240
RPT = PADN // NS


def _sc_body(h_hbm, src_hbm, dst_hbm, z_hbm, out_hbm, acc, idx_s, idx_d, rows, sem):
    cid = lax.axis_index("c")
    sid = lax.axis_index("s")
    wid = sid * NC + cid

    pltpu.sync_copy(z_hbm, acc.at[pl.ds(sid * RPT, RPT)])

    base = wid * RPW
    pltpu.sync_copy(src_hbm.at[pl.ds(base, RPW)], idx_s)
    pltpu.sync_copy(dst_hbm.at[pl.ds(base, RPW)], idx_d)
    plsc.subcore_barrier()

    def step(j, carry):
        pltpu.async_copy(h_hbm.at[idx_s.at[j]], rows, sem).wait()
        pltpu.sync_copy(rows, acc.at[idx_d.at[j]], add=True)
        return carry

    lax.fori_loop(0, RPW, step, 0)
    plsc.subcore_barrier()

    pltpu.sync_copy(acc.at[pl.ds(sid * RPT, RPT)],
                    out_hbm.at[pl.ds(cid * PADN + sid * RPT, RPT)])


@jax.jit
def _sc_segment_sum(h, src2d, dst2d, zrows):
    mesh = plsc.VectorSubcoreMesh(core_axis_name="c", subcore_axis_name="s")
    k = pl.kernel(
        _sc_body,
        out_type=jax.ShapeDtypeStruct((NC * PADN, F), jnp.float32),
        mesh=mesh,
        scratch_types=[
            pltpu.VMEM_SHARED((PADN, F), jnp.float32),
            pltpu.VMEM((RPW, IDXW), jnp.int32),
            pltpu.VMEM((RPW, IDXW), jnp.int32),
            pltpu.VMEM((IDXW, F), jnp.float32),
            pltpu.SemaphoreType.DMA,
        ],
    )
    return k(h, src2d, dst2d, zrows)


def _layer_body(h_ref, agg_ref, w1, b1, w2, b2, w3, b3, o_ref):
    z = h_ref[...] + agg_ref[0] + agg_ref[1]
    z = jnp.maximum(jnp.dot(z, w1[...], preferred_element_type=jnp.float32) + b1[...], 0.0)
    z = jnp.maximum(jnp.dot(z, w2[...], preferred_element_type=jnp.float32) + b2[...], 0.0)
    o_ref[...] = jnp.dot(z, w3[...], preferred_element_type=jnp.float32) + b3[...]


BLK = 2000
NBLK = N // BLK


def _tc_layer(h, agg2, w1, b1, w2, b2, w3, b3):
    wspec = pl.BlockSpec((F, F), lambda i: (0, 0))
    bspec = pl.BlockSpec((1, F), lambda i: (0, 0))
    return pl.pallas_call(
        _layer_body,
        grid=(NBLK,),
        in_specs=[
            pl.BlockSpec((BLK, F), lambda i: (i, 0)),
            pl.BlockSpec((NC, BLK, F), lambda i: (0, i, 0)),
            wspec, bspec, wspec, bspec, wspec, bspec,
        ],
        out_specs=pl.BlockSpec((BLK, F), lambda i: (i, 0)),
        out_shape=jax.ShapeDtypeStruct((N, F), jnp.float32),
    )(h, agg2, w1, b1, w2, b2, w3, b3)


def _pool_body(h_ref, seg_ref, d1w, d1b, d2w, d2b, o_ref, sums, cnts):
    i = pl.program_id(0)

    @pl.when(i == 0)
    def _():
        sums[...] = jnp.zeros((G, F), jnp.float32)
        cnts[...] = jnp.zeros((G, F), jnp.float32)

    onehot = (seg_ref[...] == lax.broadcasted_iota(jnp.float32, (1, G), 1)
              ).astype(jnp.float32)
    cdims = (((0,), (0,)), ((), ()))
    sums[...] += lax.dot_general(onehot, h_ref[...], cdims,
                                 preferred_element_type=jnp.float32)
    cnts[...] += lax.dot_general(onehot, jnp.ones((BLK, F), jnp.float32), cdims,
                                 preferred_element_type=jnp.float32)

    @pl.when(i == NBLK - 1)
    def _():
        pooled = sums[...] / jnp.maximum(cnts[...], 1.0)
        o1 = jnp.maximum(
            jnp.dot(pooled, d1w[...], preferred_element_type=jnp.float32) + d1b[...], 0.0)
        logits = jnp.dot(o1, d2w[...], preferred_element_type=jnp.float32) + d2b[...]
        mask = lax.broadcasted_iota(jnp.int32, (G, F), 1) < N_OUT
        logits = jnp.where(mask, logits, -1e30)
        m = jnp.max(logits, axis=1, keepdims=True)
        e = jnp.exp(logits - m)
        o_ref[...] = e / jnp.sum(e, axis=1, keepdims=True)


def _tc_pool_head(h, segf, d1w, d1b, d2wp, d2bp):
    wspec = pl.BlockSpec((F, F), lambda i: (0, 0))
    bspec = pl.BlockSpec((1, F), lambda i: (0, 0))
    return pl.pallas_call(
        _pool_body,
        grid=(NBLK,),
        in_specs=[
            pl.BlockSpec((BLK, F), lambda i: (i, 0)),
            pl.BlockSpec((BLK, 1), lambda i: (i, 0)),
            wspec, bspec, wspec, bspec,
        ],
        out_specs=pl.BlockSpec((G, F), lambda i: (0, 0)),
        out_shape=jax.ShapeDtypeStruct((G, F), jnp.float32),
        scratch_shapes=[
            pltpu.VMEM((G, F), jnp.float32),
            pltpu.VMEM((G, F), jnp.float32),
        ],
    )(h, segf, d1w, d1b, d2wp, d2bp)


def kernel(x, convW1, convb1, convW2, convb2, convW3, convb3,
           d1W, d1b, d2W, d2b, edge_index, seg):
    src = edge_index[0]
    dst = edge_index[1]
    pad = EPAD - E
    srcp = jnp.concatenate([src, jnp.zeros((pad,), jnp.int32)]).reshape(PADROWS, IDXW)
    dstp = jnp.concatenate([dst, jnp.full((pad,), N, jnp.int32)]).reshape(PADROWS, IDXW)
    zrows = jnp.zeros((RPT, F), jnp.float32)

    h = x
    for l in range(3):
        aggp = _sc_segment_sum(h, srcp, dstp, zrows)
        agg2 = aggp.reshape(NC, PADN, F)
        h = _tc_layer(h, agg2,
                      convW1[l], convb1[l].reshape(1, F),
                      convW2[l], convb2[l].reshape(1, F),
                      convW3[l], convb3[l].reshape(1, F))

    segf = seg.astype(jnp.float32).reshape(N, 1)
    d2wp = jnp.pad(d2W, ((0, 0), (0, F - N_OUT)))
    d2bp = jnp.pad(d2b, (0, F - N_OUT)).reshape(1, F)
    out = _tc_pool_head(h, segf, d1W, d1b.reshape(1, F), d2wp, d2bp)
    return out[:, :N_OUT]


# speedup vs baseline: 3.0229x; 3.0229x over previous
import functools

import jax
import jax.numpy as jnp
from jax import lax
from jax.experimental import pallas as pl
from jax.experimental.pallas import tpu as pltpu
from jax.experimental.pallas import tpu_sc as plsc

N = 10000
E = 320000
F = 128
G = 64
N_OUT = 10

NC = 2
NS = 16
NW = NC * NS

IDXW = 128
PADROWS = -(-E // (NW * IDXW * 8)) * NW * 8
EPAD = PADROWS * IDXW
RPW = PADROWS // NW
PADN = 10240
RPT = PADN // NS


def _sc_body(h_hbm, src_hbm, dst_hbm, z_hbm, out_hbm, acc, idx_s, idx_d, rows, sem):
    cid = lax.axis_index("c")
    sid = lax.axis_index("s")
    wid = sid * NC + cid

    pltpu.sync_copy(z_hbm, acc.at[pl.ds(sid * RPT, RPT)])

    base = wid * RPW
    pltpu.sync_copy(src_hbm.at[pl.ds(base, RPW)], idx_s)
    pltpu.sync_copy(dst_hbm.at[pl.ds(base, RPW)], idx_d)
    plsc.subcore_barrier()

    def step(j, carry):
        pltpu.async_copy(h_hbm.at[idx_s.at[j]], rows, sem).wait()
        pltpu.sync_copy(rows, acc.at[idx_d.at[j]], add=True)
        return carry

    lax.fori_loop(0, RPW, step, 0)
    plsc.subcore_barrier()

    pltpu.sync_copy(acc.at[pl.ds(sid * RPT, RPT)],
                    out_hbm.at[pl.ds(cid * PADN + sid * RPT, RPT)])


@jax.jit
def _sc_segment_sum(h, src2d, dst2d, zrows):
    mesh = plsc.VectorSubcoreMesh(core_axis_name="c", subcore_axis_name="s")
    k = pl.kernel(
        _sc_body,
        out_type=jax.ShapeDtypeStruct((NC * PADN, F), jnp.float32),
        mesh=mesh,
        scratch_types=[
            pltpu.VMEM_SHARED((PADN, F), jnp.float32),
            pltpu.VMEM((RPW, IDXW), jnp.int32),
            pltpu.VMEM((RPW, IDXW), jnp.int32),
            pltpu.VMEM((IDXW, F), jnp.float32),
            pltpu.SemaphoreType.DMA,
        ],
    )
    return k(h, src2d, dst2d, zrows)


def _layer_body(h_ref, agg_ref, w1, b1, w2, b2, w3, b3, o_ref):
    z = h_ref[...] + agg_ref[0] + agg_ref[1]
    z = jnp.maximum(jnp.dot(z, w1[...], preferred_element_type=jnp.float32) + b1[...], 0.0)
    z = jnp.maximum(jnp.dot(z, w2[...], preferred_element_type=jnp.float32) + b2[...], 0.0)
    o_ref[...] = jnp.dot(z, w3[...], preferred_element_type=jnp.float32) + b3[...]


BLK = 2000
NBLK = N // BLK


def _tc_layer(h, agg2, w1, b1, w2, b2, w3, b3):
    wspec = pl.BlockSpec((F, F), lambda i: (0, 0))
    bspec = pl.BlockSpec((1, F), lambda i: (0, 0))
    return pl.pallas_call(
        _layer_body,
        grid=(NBLK,),
        in_specs=[
            pl.BlockSpec((BLK, F), lambda i: (i, 0)),
            pl.BlockSpec((NC, BLK, F), lambda i: (0, i, 0)),
            wspec, bspec, wspec, bspec, wspec, bspec,
        ],
        out_specs=pl.BlockSpec((BLK, F), lambda i: (i, 0)),
        out_shape=jax.ShapeDtypeStruct((N, F), jnp.float32),
    )(h, agg2, w1, b1, w2, b2, w3, b3)


def _pool_body(h_ref, seg_ref, d1w, d1b, d2w, d2b, o_ref, sums, cnts):
    i = pl.program_id(0)

    @pl.when(i == 0)
    def _():
        sums[...] = jnp.zeros((G, F), jnp.float32)
        cnts[...] = jnp.zeros((G, F), jnp.float32)

    onehot = (seg_ref[...] == lax.broadcasted_iota(jnp.int32, (1, G), 1)
              ).astype(jnp.float32)
    cdims = (((0,), (0,)), ((), ()))
    sums[...] += lax.dot_general(onehot, h_ref[...], cdims,
                                 preferred_element_type=jnp.float32)
    cnts[...] += lax.dot_general(onehot, jnp.ones((BLK, F), jnp.float32), cdims,
                                 preferred_element_type=jnp.float32)

    @pl.when(i == NBLK - 1)
    def _():
        pooled = sums[...] / jnp.maximum(cnts[...], 1.0)
        o1 = jnp.maximum(
            jnp.dot(pooled, d1w[...], preferred_element_type=jnp.float32) + d1b[...], 0.0)
        logits = jnp.dot(o1, d2w[...], preferred_element_type=jnp.float32) + d2b[...]
        mask = lax.broadcasted_iota(jnp.int32, (G, F), 1) < N_OUT
        logits = jnp.where(mask, logits, -1e30)
        m = jnp.max(logits, axis=1, keepdims=True)
        e = jnp.exp(logits - m)
        o_ref[...] = e / jnp.sum(e, axis=1, keepdims=True)


def _tc_pool_head(h, segf, d1w, d1b, d2wp, d2bp):
    wspec = pl.BlockSpec((F, F), lambda i: (0, 0))
    bspec = pl.BlockSpec((1, F), lambda i: (0, 0))
    return pl.pallas_call(
        _pool_body,
        grid=(NBLK,),
        in_specs=[
            pl.BlockSpec((BLK, F), lambda i: (i, 0)),
            pl.BlockSpec((BLK, 1), lambda i: (i, 0)),
            wspec, bspec, wspec, bspec,
        ],
        out_specs=pl.BlockSpec((G, F), lambda i: (0, 0)),
        out_shape=jax.ShapeDtypeStruct((G, F), jnp.float32),
        scratch_shapes=[
            pltpu.VMEM((G, F), jnp.float32),
            pltpu.VMEM((G, F), jnp.float32),
        ],
    )(h, segf, d1w, d1b, d2wp, d2bp)


def kernel(x, convW1, convb1, convW2, convb2, convW3, convb3,
           d1W, d1b, d2W, d2b, edge_index, seg):
    src = edge_index[0]
    dst = edge_index[1]
    pad = EPAD - E
    srcp = jnp.concatenate([src, jnp.zeros((pad,), jnp.int32)]).reshape(PADROWS, IDXW)
    dstp = jnp.concatenate([dst, jnp.full((pad,), N, jnp.int32)]).reshape(PADROWS, IDXW)
    zrows = jnp.zeros((RPT, F), jnp.float32)

    h = x
    for l in range(3):
        aggp = _sc_segment_sum(h, srcp, dstp, zrows)
        agg2 = aggp.reshape(NC, PADN, F)
        h = _tc_layer(h, agg2,
                      convW1[l], convb1[l].reshape(1, F),
                      convW2[l], convb2[l].reshape(1, F),
                      convW3[l], convb3[l].reshape(1, F))

    segf = seg.reshape(N, 1)
    d2wp = jnp.pad(d2W, ((0, 0), (0, F - N_OUT)))
    d2bp = jnp.pad(d2b, (0, F - N_OUT)).reshape(1, F)
    out = _tc_pool_head(h, segf, d1W, d1b.reshape(1, F), d2wp, d2bp)
    return out[:, :N_OUT]


# speedup vs baseline: 3.4404x; 1.1381x over previous
import functools

import jax
import jax.numpy as jnp
from jax import lax
from jax.experimental import pallas as pl
from jax.experimental.pallas import tpu as pltpu
from jax.experimental.pallas import tpu_sc as plsc

N = 10000
E = 320000
F = 128
G = 64
N_OUT = 10

NC = 2
NS = 16
NW = NC * NS

IDXW = 128
PADROWS = -(-E // (NW * IDXW * 8)) * NW * 8
EPAD = PADROWS * IDXW
RPW = PADROWS // NW
PADN = 10240
RPT = PADN // NS
HALF = RPW // 2


def _sc_body(h_hbm, src_hbm, dst_hbm, z_hbm, out_hbm, acc, idx_s, idx_d,
             rows0, rows1, sem0, sem1):
    cid = lax.axis_index("c")
    sid = lax.axis_index("s")
    wid = sid * NC + cid

    pltpu.sync_copy(z_hbm, acc.at[pl.ds(sid * RPT, RPT)])

    base = wid * RPW
    plsc.subcore_barrier()

    for hi in range(2):
        hbase = base + hi * HALF
        pltpu.sync_copy(src_hbm.at[pl.ds(hbase, HALF)], idx_s)
        pltpu.sync_copy(dst_hbm.at[pl.ds(hbase, HALF)], idx_d)
        pltpu.async_copy(h_hbm.at[idx_s.at[0]], rows0, sem0)
        pltpu.async_copy(h_hbm.at[idx_s.at[1]], rows1, sem1)

        def step(i, carry):
            j = 2 * i
            for off, rows, sem in ((0, rows0, sem0), (1, rows1, sem1)):
                pltpu.make_async_copy(h_hbm.at[idx_s.at[j + off]], rows, sem).wait()
                pltpu.sync_copy(rows, acc.at[idx_d.at[j + off]], add=True)

                @pl.when(j + off + 2 < HALF)
                def _():
                    pltpu.async_copy(h_hbm.at[idx_s.at[j + off + 2]], rows, sem)
            return carry

        lax.fori_loop(0, HALF // 2, step, 0)
    plsc.subcore_barrier()

    pltpu.sync_copy(acc.at[pl.ds(sid * RPT, RPT)],
                    out_hbm.at[pl.ds(cid * PADN + sid * RPT, RPT)])


@jax.jit
def _sc_segment_sum(h, src2d, dst2d, zrows):
    mesh = plsc.VectorSubcoreMesh(core_axis_name="c", subcore_axis_name="s")
    k = pl.kernel(
        _sc_body,
        out_type=jax.ShapeDtypeStruct((NC * PADN, F), jnp.float32),
        mesh=mesh,
        scratch_types=[
            pltpu.VMEM_SHARED((PADN, F), jnp.float32),
            pltpu.VMEM((HALF, IDXW), jnp.int32),
            pltpu.VMEM((HALF, IDXW), jnp.int32),
            pltpu.VMEM((IDXW, F), jnp.float32),
            pltpu.VMEM((IDXW, F), jnp.float32),
            pltpu.SemaphoreType.DMA,
            pltpu.SemaphoreType.DMA,
        ],
    )
    return k(h, src2d, dst2d, zrows)


def _layer_body(h_ref, agg_ref, w1, b1, w2, b2, w3, b3, o_ref):
    z = h_ref[...] + agg_ref[0] + agg_ref[1]
    z = jnp.maximum(jnp.dot(z, w1[...], preferred_element_type=jnp.float32) + b1[...], 0.0)
    z = jnp.maximum(jnp.dot(z, w2[...], preferred_element_type=jnp.float32) + b2[...], 0.0)
    o_ref[...] = jnp.dot(z, w3[...], preferred_element_type=jnp.float32) + b3[...]


BLK = 2000
NBLK = N // BLK


def _tc_layer(h, agg2, w1, b1, w2, b2, w3, b3):
    wspec = pl.BlockSpec((F, F), lambda i: (0, 0))
    bspec = pl.BlockSpec((1, F), lambda i: (0, 0))
    return pl.pallas_call(
        _layer_body,
        grid=(NBLK,),
        in_specs=[
            pl.BlockSpec((BLK, F), lambda i: (i, 0)),
            pl.BlockSpec((NC, BLK, F), lambda i: (0, i, 0)),
            wspec, bspec, wspec, bspec, wspec, bspec,
        ],
        out_specs=pl.BlockSpec((BLK, F), lambda i: (i, 0)),
        out_shape=jax.ShapeDtypeStruct((N, F), jnp.float32),
    )(h, agg2, w1, b1, w2, b2, w3, b3)


def _pool_body(h_ref, seg_ref, d1w, d1b, d2w, d2b, o_ref, sums, cnts):
    i = pl.program_id(0)

    @pl.when(i == 0)
    def _():
        sums[...] = jnp.zeros((G, F), jnp.float32)
        cnts[...] = jnp.zeros((G, F), jnp.float32)

    onehot = (seg_ref[...] == lax.broadcasted_iota(jnp.int32, (1, G), 1)
              ).astype(jnp.float32)
    cdims = (((0,), (0,)), ((), ()))
    sums[...] += lax.dot_general(onehot, h_ref[...], cdims,
                                 preferred_element_type=jnp.float32)
    cnts[...] += lax.dot_general(onehot, jnp.ones((BLK, F), jnp.float32), cdims,
                                 preferred_element_type=jnp.float32)

    @pl.when(i == NBLK - 1)
    def _():
        pooled = sums[...] / jnp.maximum(cnts[...], 1.0)
        o1 = jnp.maximum(
            jnp.dot(pooled, d1w[...], preferred_element_type=jnp.float32) + d1b[...], 0.0)
        logits = jnp.dot(o1, d2w[...], preferred_element_type=jnp.float32) + d2b[...]
        mask = lax.broadcasted_iota(jnp.int32, (G, F), 1) < N_OUT
        logits = jnp.where(mask, logits, -1e30)
        m = jnp.max(logits, axis=1, keepdims=True)
        e = jnp.exp(logits - m)
        o_ref[...] = e / jnp.sum(e, axis=1, keepdims=True)


def _tc_pool_head(h, segf, d1w, d1b, d2wp, d2bp):
    wspec = pl.BlockSpec((F, F), lambda i: (0, 0))
    bspec = pl.BlockSpec((1, F), lambda i: (0, 0))
    return pl.pallas_call(
        _pool_body,
        grid=(NBLK,),
        in_specs=[
            pl.BlockSpec((BLK, F), lambda i: (i, 0)),
            pl.BlockSpec((BLK, 1), lambda i: (i, 0)),
            wspec, bspec, wspec, bspec,
        ],
        out_specs=pl.BlockSpec((G, F), lambda i: (0, 0)),
        out_shape=jax.ShapeDtypeStruct((G, F), jnp.float32),
        scratch_shapes=[
            pltpu.VMEM((G, F), jnp.float32),
            pltpu.VMEM((G, F), jnp.float32),
        ],
    )(h, segf, d1w, d1b, d2wp, d2bp)


def kernel(x, convW1, convb1, convW2, convb2, convW3, convb3,
           d1W, d1b, d2W, d2b, edge_index, seg):
    src = edge_index[0]
    dst = edge_index[1]
    pad = EPAD - E
    srcp = jnp.concatenate([src, jnp.zeros((pad,), jnp.int32)]).reshape(PADROWS, IDXW)
    dstp = jnp.concatenate([dst, jnp.full((pad,), N, jnp.int32)]).reshape(PADROWS, IDXW)
    zrows = jnp.zeros((RPT, F), jnp.float32)

    h = x
    for l in range(3):
        aggp = _sc_segment_sum(h, srcp, dstp, zrows)
        agg2 = aggp.reshape(NC, PADN, F)
        h = _tc_layer(h, agg2,
                      convW1[l], convb1[l].reshape(1, F),
                      convW2[l], convb2[l].reshape(1, F),
                      convW3[l], convb3[l].reshape(1, F))

    segf = seg.reshape(N, 1)
    d2wp = jnp.pad(d2W, ((0, 0), (0, F - N_OUT)))
    d2bp = jnp.pad(d2b, (0, F - N_OUT)).reshape(1, F)
    out = _tc_pool_head(h, segf, d1W, d1b.reshape(1, F), d2wp, d2bp)
    return out[:, :N_OUT]


# speedup vs baseline: 3.6334x; 1.0561x over previous
import functools

import jax
import jax.numpy as jnp
from jax import lax
from jax.experimental import pallas as pl
from jax.experimental.pallas import tpu as pltpu
from jax.experimental.pallas import tpu_sc as plsc

N = 10000
E = 320000
F = 128
G = 64
N_OUT = 10

NC = 2
NS = 16
NW = NC * NS

IDXW = 128
PADROWS = -(-E // (NW * IDXW * 8)) * NW * 8
EPAD = PADROWS * IDXW
RPW = PADROWS // NW
PADN = 10240
RPT = PADN // NS
CH = 40
R0 = 120
R1 = 2 * RPW - R0


def _sc_body(h_hbm, src_hbm, dst_hbm, z_hbm, out_hbm, acc, idx_s, idx_d,
             rows0, rows1, sem0, sem1):
    cid = lax.axis_index("c")
    sid = lax.axis_index("s")
    wid = sid * NC + cid

    pltpu.sync_copy(z_hbm, acc.at[pl.ds(sid * RPT, RPT)])

    base = jnp.where(cid == 0, sid * R0, NS * R0 + sid * R1)
    nchunks = jnp.where(cid == 0, R0 // CH, R1 // CH)
    plsc.subcore_barrier()

    def chunk(ci, carry):
        hbase = base + ci * CH
        pltpu.sync_copy(src_hbm.at[pl.ds(hbase, CH)], idx_s)
        pltpu.sync_copy(dst_hbm.at[pl.ds(hbase, CH)], idx_d)
        pltpu.async_copy(h_hbm.at[idx_s.at[0]], rows0, sem0)
        pltpu.async_copy(h_hbm.at[idx_s.at[1]], rows1, sem1)

        def step(i, c):
            j = 2 * i
            for off, rows, sem in ((0, rows0, sem0), (1, rows1, sem1)):
                pltpu.make_async_copy(h_hbm.at[idx_s.at[j + off]], rows, sem).wait()
                pltpu.sync_copy(rows, acc.at[idx_d.at[j + off]], add=True)

                @pl.when(j + off + 2 < CH)
                def _():
                    pltpu.async_copy(h_hbm.at[idx_s.at[j + off + 2]], rows, sem)
            return c

        return lax.fori_loop(0, CH // 2, step, carry)

    lax.fori_loop(0, nchunks, chunk, 0)
    plsc.subcore_barrier()

    pltpu.sync_copy(acc.at[pl.ds(sid * RPT, RPT)],
                    out_hbm.at[pl.ds(cid * PADN + sid * RPT, RPT)])


@jax.jit
def _sc_segment_sum(h, src2d, dst2d, zrows):
    mesh = plsc.VectorSubcoreMesh(core_axis_name="c", subcore_axis_name="s")
    k = pl.kernel(
        _sc_body,
        out_type=jax.ShapeDtypeStruct((NC * PADN, F), jnp.float32),
        mesh=mesh,
        scratch_types=[
            pltpu.VMEM_SHARED((PADN, F), jnp.float32),
            pltpu.VMEM((CH, IDXW), jnp.int32),
            pltpu.VMEM((CH, IDXW), jnp.int32),
            pltpu.VMEM((IDXW, F), jnp.float32),
            pltpu.VMEM((IDXW, F), jnp.float32),
            pltpu.SemaphoreType.DMA,
            pltpu.SemaphoreType.DMA,
        ],
    )
    return k(h, src2d, dst2d, zrows)


def _layer_body(h_ref, agg_ref, w1, b1, w2, b2, w3, b3, o_ref):
    z = h_ref[...] + agg_ref[0] + agg_ref[1]
    z = jnp.maximum(jnp.dot(z, w1[...], preferred_element_type=jnp.float32) + b1[...], 0.0)
    z = jnp.maximum(jnp.dot(z, w2[...], preferred_element_type=jnp.float32) + b2[...], 0.0)
    o_ref[...] = jnp.dot(z, w3[...], preferred_element_type=jnp.float32) + b3[...]


BLK = 2000
NBLK = N // BLK


def _tc_layer(h, agg2, w1, b1, w2, b2, w3, b3):
    wspec = pl.BlockSpec((F, F), lambda i: (0, 0))
    bspec = pl.BlockSpec((1, F), lambda i: (0, 0))
    return pl.pallas_call(
        _layer_body,
        grid=(NBLK,),
        in_specs=[
            pl.BlockSpec((BLK, F), lambda i: (i, 0)),
            pl.BlockSpec((NC, BLK, F), lambda i: (0, i, 0)),
            wspec, bspec, wspec, bspec, wspec, bspec,
        ],
        out_specs=pl.BlockSpec((BLK, F), lambda i: (i, 0)),
        out_shape=jax.ShapeDtypeStruct((N, F), jnp.float32),
    )(h, agg2, w1, b1, w2, b2, w3, b3)


def _pool_body(h_ref, seg_ref, d1w, d1b, d2w, d2b, o_ref, sums, cnts):
    i = pl.program_id(0)

    @pl.when(i == 0)
    def _():
        sums[...] = jnp.zeros((G, F), jnp.float32)
        cnts[...] = jnp.zeros((G, F), jnp.float32)

    onehot = (seg_ref[...] == lax.broadcasted_iota(jnp.int32, (1, G), 1)
              ).astype(jnp.float32)
    cdims = (((0,), (0,)), ((), ()))
    sums[...] += lax.dot_general(onehot, h_ref[...], cdims,
                                 preferred_element_type=jnp.float32)
    cnts[...] += lax.dot_general(onehot, jnp.ones((BLK, F), jnp.float32), cdims,
                                 preferred_element_type=jnp.float32)

    @pl.when(i == NBLK - 1)
    def _():
        pooled = sums[...] / jnp.maximum(cnts[...], 1.0)
        o1 = jnp.maximum(
            jnp.dot(pooled, d1w[...], preferred_element_type=jnp.float32) + d1b[...], 0.0)
        logits = jnp.dot(o1, d2w[...], preferred_element_type=jnp.float32) + d2b[...]
        mask = lax.broadcasted_iota(jnp.int32, (G, F), 1) < N_OUT
        logits = jnp.where(mask, logits, -1e30)
        m = jnp.max(logits, axis=1, keepdims=True)
        e = jnp.exp(logits - m)
        o_ref[...] = e / jnp.sum(e, axis=1, keepdims=True)


def _tc_pool_head(h, segf, d1w, d1b, d2wp, d2bp):
    wspec = pl.BlockSpec((F, F), lambda i: (0, 0))
    bspec = pl.BlockSpec((1, F), lambda i: (0, 0))
    return pl.pallas_call(
        _pool_body,
        grid=(NBLK,),
        in_specs=[
            pl.BlockSpec((BLK, F), lambda i: (i, 0)),
            pl.BlockSpec((BLK, 1), lambda i: (i, 0)),
            wspec, bspec, wspec, bspec,
        ],
        out_specs=pl.BlockSpec((G, F), lambda i: (0, 0)),
        out_shape=jax.ShapeDtypeStruct((G, F), jnp.float32),
        scratch_shapes=[
            pltpu.VMEM((G, F), jnp.float32),
            pltpu.VMEM((G, F), jnp.float32),
        ],
    )(h, segf, d1w, d1b, d2wp, d2bp)


def kernel(x, convW1, convb1, convW2, convb2, convW3, convb3,
           d1W, d1b, d2W, d2b, edge_index, seg):
    src = edge_index[0]
    dst = edge_index[1]
    pad = EPAD - E
    srcp = jnp.concatenate([src, jnp.zeros((pad,), jnp.int32)]).reshape(PADROWS, IDXW)
    dstp = jnp.concatenate([dst, jnp.full((pad,), N, jnp.int32)]).reshape(PADROWS, IDXW)
    zrows = jnp.zeros((RPT, F), jnp.float32)

    h = x
    for l in range(3):
        aggp = _sc_segment_sum(h, srcp, dstp, zrows)
        agg2 = aggp.reshape(NC, PADN, F)
        h = _tc_layer(h, agg2,
                      convW1[l], convb1[l].reshape(1, F),
                      convW2[l], convb2[l].reshape(1, F),
                      convW3[l], convb3[l].reshape(1, F))

    segf = seg.reshape(N, 1)
    d2wp = jnp.pad(d2W, ((0, 0), (0, F - N_OUT)))
    d2bp = jnp.pad(d2b, (0, F - N_OUT)).reshape(1, F)
    out = _tc_pool_head(h, segf, d1W, d1b.reshape(1, F), d2wp, d2bp)
    return out[:, :N_OUT]


# speedup vs baseline: 3.6335x; 1.0000x over previous
import functools

import jax
import jax.numpy as jnp
from jax import lax
from jax.experimental import pallas as pl
from jax.experimental.pallas import tpu as pltpu
from jax.experimental.pallas import tpu_sc as plsc

N = 10000
E = 320000
F = 128
G = 64
N_OUT = 10

NC = 2
NS = 16
NW = NC * NS

IDXW = 128
PADROWS = -(-E // (NW * IDXW * 8)) * NW * 8
EPAD = PADROWS * IDXW
RPW = PADROWS // NW
PADN = 10240
RPT = PADN // NS
CH = 40
R0 = 120
R1 = 2 * RPW - R0


def _sc_body(h_hbm, src_hbm, dst_hbm, z_hbm, out_hbm, acc, idx_s, idx_d,
             rows0, rows1, sem0, sem1):
    cid = lax.axis_index("c")
    sid = lax.axis_index("s")
    wid = sid * NC + cid

    with jax.named_scope("ph_zero"):
        pltpu.sync_copy(z_hbm, acc.at[pl.ds(sid * RPT, RPT)])

    base = jnp.where(cid == 0, sid * R0, NS * R0 + sid * R1)
    nchunks = jnp.where(cid == 0, R0 // CH, R1 // CH)
    plsc.subcore_barrier()

    def chunk(ci, carry):
        hbase = base + ci * CH
        pltpu.sync_copy(src_hbm.at[pl.ds(hbase, CH)], idx_s)
        pltpu.sync_copy(dst_hbm.at[pl.ds(hbase, CH)], idx_d)
        pltpu.async_copy(h_hbm.at[idx_s.at[0]], rows0, sem0)
        pltpu.async_copy(h_hbm.at[idx_s.at[1]], rows1, sem1)

        def step(i, c):
            j = 2 * i
            for off, rows, sem in ((0, rows0, sem0), (1, rows1, sem1)):
                pltpu.make_async_copy(h_hbm.at[idx_s.at[j + off]], rows, sem).wait()
                pltpu.sync_copy(rows, acc.at[idx_d.at[j + off]], add=True)

                @pl.when(j + off + 2 < CH)
                def _():
                    pltpu.async_copy(h_hbm.at[idx_s.at[j + off + 2]], rows, sem)
            return c

        return lax.fori_loop(0, CH // 2, step, carry)

    with jax.named_scope("ph_edges"):
        lax.fori_loop(0, nchunks, chunk, 0)
    with jax.named_scope("ph_bar"):
        plsc.subcore_barrier()

    with jax.named_scope("ph_writeout"):
        pltpu.sync_copy(acc.at[pl.ds(sid * RPT, RPT)],
                        out_hbm.at[pl.ds(cid * PADN + sid * RPT, RPT)])


@jax.jit
def _sc_segment_sum(h, src2d, dst2d, zrows):
    mesh = plsc.VectorSubcoreMesh(core_axis_name="c", subcore_axis_name="s")
    k = pl.kernel(
        _sc_body,
        out_type=jax.ShapeDtypeStruct((NC * PADN, F), jnp.float32),
        mesh=mesh,
        scratch_types=[
            pltpu.VMEM_SHARED((PADN, F), jnp.float32),
            pltpu.VMEM((CH, IDXW), jnp.int32),
            pltpu.VMEM((CH, IDXW), jnp.int32),
            pltpu.VMEM((IDXW, F), jnp.float32),
            pltpu.VMEM((IDXW, F), jnp.float32),
            pltpu.SemaphoreType.DMA,
            pltpu.SemaphoreType.DMA,
        ],
    )
    return k(h, src2d, dst2d, zrows)


def _layer_body(h_ref, agg_ref, w1, b1, w2, b2, w3, b3, o_ref):
    z = h_ref[...] + agg_ref[0] + agg_ref[1]
    z = jnp.maximum(jnp.dot(z, w1[...], preferred_element_type=jnp.float32) + b1[...], 0.0)
    z = jnp.maximum(jnp.dot(z, w2[...], preferred_element_type=jnp.float32) + b2[...], 0.0)
    o_ref[...] = jnp.dot(z, w3[...], preferred_element_type=jnp.float32) + b3[...]


BLK = 2000
NBLK = N // BLK


def _tc_layer(h, agg2, w1, b1, w2, b2, w3, b3):
    wspec = pl.BlockSpec((F, F), lambda i: (0, 0))
    bspec = pl.BlockSpec((1, F), lambda i: (0, 0))
    return pl.pallas_call(
        _layer_body,
        grid=(NBLK,),
        in_specs=[
            pl.BlockSpec((BLK, F), lambda i: (i, 0)),
            pl.BlockSpec((NC, BLK, F), lambda i: (0, i, 0)),
            wspec, bspec, wspec, bspec, wspec, bspec,
        ],
        out_specs=pl.BlockSpec((BLK, F), lambda i: (i, 0)),
        out_shape=jax.ShapeDtypeStruct((N, F), jnp.float32),
    )(h, agg2, w1, b1, w2, b2, w3, b3)


def _pool_body(h_ref, seg_ref, d1w, d1b, d2w, d2b, o_ref, sums, cnts):
    i = pl.program_id(0)

    @pl.when(i == 0)
    def _():
        sums[...] = jnp.zeros((G, F), jnp.float32)
        cnts[...] = jnp.zeros((G, F), jnp.float32)

    onehot = (seg_ref[...] == lax.broadcasted_iota(jnp.int32, (1, G), 1)
              ).astype(jnp.float32)
    cdims = (((0,), (0,)), ((), ()))
    sums[...] += lax.dot_general(onehot, h_ref[...], cdims,
                                 preferred_element_type=jnp.float32)
    cnts[...] += lax.dot_general(onehot, jnp.ones((BLK, F), jnp.float32), cdims,
                                 preferred_element_type=jnp.float32)

    @pl.when(i == NBLK - 1)
    def _():
        pooled = sums[...] / jnp.maximum(cnts[...], 1.0)
        o1 = jnp.maximum(
            jnp.dot(pooled, d1w[...], preferred_element_type=jnp.float32) + d1b[...], 0.0)
        logits = jnp.dot(o1, d2w[...], preferred_element_type=jnp.float32) + d2b[...]
        mask = lax.broadcasted_iota(jnp.int32, (G, F), 1) < N_OUT
        logits = jnp.where(mask, logits, -1e30)
        m = jnp.max(logits, axis=1, keepdims=True)
        e = jnp.exp(logits - m)
        o_ref[...] = e / jnp.sum(e, axis=1, keepdims=True)


def _tc_pool_head(h, segf, d1w, d1b, d2wp, d2bp):
    wspec = pl.BlockSpec((F, F), lambda i: (0, 0))
    bspec = pl.BlockSpec((1, F), lambda i: (0, 0))
    return pl.pallas_call(
        _pool_body,
        grid=(NBLK,),
        in_specs=[
            pl.BlockSpec((BLK, F), lambda i: (i, 0)),
            pl.BlockSpec((BLK, 1), lambda i: (i, 0)),
            wspec, bspec, wspec, bspec,
        ],
        out_specs=pl.BlockSpec((G, F), lambda i: (0, 0)),
        out_shape=jax.ShapeDtypeStruct((G, F), jnp.float32),
        scratch_shapes=[
            pltpu.VMEM((G, F), jnp.float32),
            pltpu.VMEM((G, F), jnp.float32),
        ],
    )(h, segf, d1w, d1b, d2wp, d2bp)


def kernel(x, convW1, convb1, convW2, convb2, convW3, convb3,
           d1W, d1b, d2W, d2b, edge_index, seg):
    src = edge_index[0]
    dst = edge_index[1]
    pad = EPAD - E
    srcp = jnp.concatenate([src, jnp.zeros((pad,), jnp.int32)]).reshape(PADROWS, IDXW)
    dstp = jnp.concatenate([dst, jnp.full((pad,), N, jnp.int32)]).reshape(PADROWS, IDXW)
    zrows = jnp.zeros((RPT, F), jnp.float32)

    h = x
    for l in range(3):
        aggp = _sc_segment_sum(h, srcp, dstp, zrows)
        agg2 = aggp.reshape(NC, PADN, F)
        h = _tc_layer(h, agg2,
                      convW1[l], convb1[l].reshape(1, F),
                      convW2[l], convb2[l].reshape(1, F),
                      convW3[l], convb3[l].reshape(1, F))

    segf = seg.reshape(N, 1)
    d2wp = jnp.pad(d2W, ((0, 0), (0, F - N_OUT)))
    d2bp = jnp.pad(d2b, (0, F - N_OUT)).reshape(1, F)
    out = _tc_pool_head(h, segf, d1W, d1b.reshape(1, F), d2wp, d2bp)
    return out[:, :N_OUT]


# speedup vs baseline: 11.3351x; 3.1196x over previous
import functools

import jax
import jax.numpy as jnp
from jax import lax
from jax.experimental import pallas as pl
from jax.experimental.pallas import tpu as pltpu
from jax.experimental.pallas import tpu_sc as plsc

N = 10000
E = 320000
F = 128
G = 64
N_OUT = 10

NC = 2
NS = 16
NW = NC * NS

IDXW = 128
PADROWS = -(-E // (NW * IDXW * 8)) * NW * 8
EPAD = PADROWS * IDXW
RPW = PADROWS // NW
PADN = 10240
RPT = PADN // NS
CH = 40
R0 = RPW
R1 = 2 * RPW - R0


def _sc_body(h_hbm, src_hbm, dst_hbm, z_hbm, out_hbm, acc, idx_s, idx_d,
             rows0, rows1, sem0, sem1):
    cid = lax.axis_index("c")
    sid = lax.axis_index("s")
    wid = sid * NC + cid

    with jax.named_scope("ph_zero"):
        pltpu.sync_copy(z_hbm, acc.at[pl.ds(sid * RPT, RPT)])

    base = jnp.where(cid == 0, sid * R0, NS * R0 + sid * R1)
    nchunks = jnp.where(cid == 0, R0 // CH, R1 // CH)
    plsc.subcore_barrier()

    def chunk(ci, carry):
        hbase = base + ci * CH
        pltpu.sync_copy(src_hbm.at[pl.ds(hbase, CH)], idx_s)
        pltpu.sync_copy(dst_hbm.at[pl.ds(hbase, CH)], idx_d)
        pltpu.async_copy(h_hbm.at[idx_s.at[0]], rows0, sem0)
        pltpu.async_copy(h_hbm.at[idx_s.at[1]], rows1, sem1)

        def step(i, c):
            j = 2 * i
            for off, rows, sem in ((0, rows0, sem0), (1, rows1, sem1)):
                pltpu.make_async_copy(h_hbm.at[idx_s.at[j + off]], rows, sem).wait()
                pltpu.sync_copy(rows, acc.at[idx_d.at[j + off]], add=True)

                @pl.when(j + off + 2 < CH)
                def _():
                    pltpu.async_copy(h_hbm.at[idx_s.at[j + off + 2]], rows, sem)
            return c

        return lax.fori_loop(0, CH // 2, step, carry)

    with jax.named_scope("ph_edges"):
        lax.fori_loop(0, nchunks, chunk, 0)
    with jax.named_scope("ph_bar"):
        plsc.subcore_barrier()

    with jax.named_scope("ph_writeout"):
        pltpu.sync_copy(acc.at[pl.ds(sid * RPT, RPT)],
                        out_hbm.at[pl.ds(cid * PADN + sid * RPT, RPT)])


@jax.jit
def _sc_segment_sum(h, src2d, dst2d, zrows):
    mesh = plsc.VectorSubcoreMesh(core_axis_name="c", subcore_axis_name="s")
    k = pl.kernel(
        _sc_body,
        out_type=jax.ShapeDtypeStruct((NC * PADN, F), jnp.float32),
        mesh=mesh,
        scratch_types=[
            pltpu.VMEM_SHARED((PADN, F), jnp.float32),
            pltpu.VMEM((CH, IDXW), jnp.int32),
            pltpu.VMEM((CH, IDXW), jnp.int32),
            pltpu.VMEM((IDXW, F), jnp.float32),
            pltpu.VMEM((IDXW, F), jnp.float32),
            pltpu.SemaphoreType.DMA,
            pltpu.SemaphoreType.DMA,
        ],
    )
    return k(h, src2d, dst2d, zrows)


def _layer_body(h_ref, agg_ref, w1, b1, w2, b2, w3, b3, o_ref):
    z = h_ref[...] + agg_ref[0] + agg_ref[1]
    z = jnp.maximum(jnp.dot(z, w1[...], preferred_element_type=jnp.float32) + b1[...], 0.0)
    z = jnp.maximum(jnp.dot(z, w2[...], preferred_element_type=jnp.float32) + b2[...], 0.0)
    o_ref[...] = jnp.dot(z, w3[...], preferred_element_type=jnp.float32) + b3[...]


BLK = 2000
NBLK = N // BLK


def _tc_layer(h, agg2, w1, b1, w2, b2, w3, b3):
    wspec = pl.BlockSpec((F, F), lambda i: (0, 0))
    bspec = pl.BlockSpec((1, F), lambda i: (0, 0))
    return pl.pallas_call(
        _layer_body,
        grid=(NBLK,),
        in_specs=[
            pl.BlockSpec((BLK, F), lambda i: (i, 0)),
            pl.BlockSpec((NC, BLK, F), lambda i: (0, i, 0)),
            wspec, bspec, wspec, bspec, wspec, bspec,
        ],
        out_specs=pl.BlockSpec((BLK, F), lambda i: (i, 0)),
        out_shape=jax.ShapeDtypeStruct((N, F), jnp.float32),
    )(h, agg2, w1, b1, w2, b2, w3, b3)


def _pool_body(h_ref, seg_ref, d1w, d1b, d2w, d2b, o_ref, sums, cnts):
    i = pl.program_id(0)

    @pl.when(i == 0)
    def _():
        sums[...] = jnp.zeros((G, F), jnp.float32)
        cnts[...] = jnp.zeros((G, F), jnp.float32)

    onehot = (seg_ref[...] == lax.broadcasted_iota(jnp.int32, (1, G), 1)
              ).astype(jnp.float32)
    cdims = (((0,), (0,)), ((), ()))
    sums[...] += lax.dot_general(onehot, h_ref[...], cdims,
                                 preferred_element_type=jnp.float32)
    cnts[...] += lax.dot_general(onehot, jnp.ones((BLK, F), jnp.float32), cdims,
                                 preferred_element_type=jnp.float32)

    @pl.when(i == NBLK - 1)
    def _():
        pooled = sums[...] / jnp.maximum(cnts[...], 1.0)
        o1 = jnp.maximum(
            jnp.dot(pooled, d1w[...], preferred_element_type=jnp.float32) + d1b[...], 0.0)
        logits = jnp.dot(o1, d2w[...], preferred_element_type=jnp.float32) + d2b[...]
        mask = lax.broadcasted_iota(jnp.int32, (G, F), 1) < N_OUT
        logits = jnp.where(mask, logits, -1e30)
        m = jnp.max(logits, axis=1, keepdims=True)
        e = jnp.exp(logits - m)
        o_ref[...] = e / jnp.sum(e, axis=1, keepdims=True)


def _tc_pool_head(h, segf, d1w, d1b, d2wp, d2bp):
    wspec = pl.BlockSpec((F, F), lambda i: (0, 0))
    bspec = pl.BlockSpec((1, F), lambda i: (0, 0))
    return pl.pallas_call(
        _pool_body,
        grid=(NBLK,),
        in_specs=[
            pl.BlockSpec((BLK, F), lambda i: (i, 0)),
            pl.BlockSpec((BLK, 1), lambda i: (i, 0)),
            wspec, bspec, wspec, bspec,
        ],
        out_specs=pl.BlockSpec((G, F), lambda i: (0, 0)),
        out_shape=jax.ShapeDtypeStruct((G, F), jnp.float32),
        scratch_shapes=[
            pltpu.VMEM((G, F), jnp.float32),
            pltpu.VMEM((G, F), jnp.float32),
        ],
    )(h, segf, d1w, d1b, d2wp, d2bp)


def kernel(x, convW1, convb1, convW2, convb2, convW3, convb3,
           d1W, d1b, d2W, d2b, edge_index, seg):
    src = edge_index[0]
    dst = edge_index[1]
    pad = EPAD - E
    ar = jnp.arange(pad, dtype=jnp.int32)
    srcp = jnp.concatenate([src, ar % N]).reshape(PADROWS, IDXW)
    dstp = jnp.concatenate([dst, N + ar % (PADN - N)]).reshape(PADROWS, IDXW)
    zrows = jnp.zeros((RPT, F), jnp.float32)

    h = x
    for l in range(3):
        aggp = _sc_segment_sum(h, srcp, dstp, zrows)
        agg2 = aggp.reshape(NC, PADN, F)
        h = _tc_layer(h, agg2,
                      convW1[l], convb1[l].reshape(1, F),
                      convW2[l], convb2[l].reshape(1, F),
                      convW3[l], convb3[l].reshape(1, F))

    segf = seg.reshape(N, 1)
    d2wp = jnp.pad(d2W, ((0, 0), (0, F - N_OUT)))
    d2bp = jnp.pad(d2b, (0, F - N_OUT)).reshape(1, F)
    out = _tc_pool_head(h, segf, d1W, d1b.reshape(1, F), d2wp, d2bp)
    return out[:, :N_OUT]
